# baseline (speedup 1.0000x reference)
import functools

import jax
import jax.numpy as jnp
import numpy as np
from jax import lax
from jax.experimental import pallas as pl
from jax.experimental.pallas import tpu as pltpu
from jax.experimental.pallas import tpu_sc as plsc

N = 10000
E = 320000
F32 = jnp.float32
I32 = jnp.int32

NRANGE = 4
NQ = 2500
ACCR = 2560
RPT0 = 160
EPT0 = E // 16
BLKE = 2000
NSUB0 = EPT0 // BLKE
NPAD0 = NRANGE * NQ + 16

ACC1R = 10112
RPT1 = 632
EPT1 = E // 32
NBLK1 = EPT1 // BLKE
NCH1 = (EPT1 + 16) // 32

_Q = np.arange(512)
_HH = _Q % 8
_EYEA = np.zeros((512, 16), np.float32)
_EYEA[_Q, _HH] = 1.0
_EYEA[_Q, _HH + 8] = 1.0
_P2 = np.arange(512).reshape(16, 16, 2).transpose(0, 2, 1).reshape(512)
_HH2 = _HH[_P2]
_SIL = np.zeros((16, 512), np.float32)
_SIL[_HH2, _Q] = 1.0
_IDXE = (2 * np.arange(16)) % 8
_IDXO = (2 * np.arange(16) + 1) % 8
_S1 = np.zeros((16, 48), np.float32)
_S1[0, :] = 1.0


def _edge0_body(feat_hbm, elrs_hbm, elrd_hbm, src_hbm, dst_hbm,
                num_out, den_out,
                stage_src, stage_dst, csrc, cdst, didx,
                fbuf, sbuf, esbuf, edbuf, wbuf, zbuf, zbuf16,
                acc, den, gsem, ssem):
  cid = lax.axis_index("c")
  t = lax.axis_index("s")
  iota16 = lax.iota(I32, 16)
  zero16 = jnp.zeros((16,), F32)

  def _zb(r, c):
    for j in range(32):
      zbuf[r, pl.ds(j * 16, 16)] = zero16
    return c
  lax.fori_loop(0, 8, _zb, 0)

  def _zb16(r, c):
    zbuf16[r, :] = zero16
    return c
  lax.fori_loop(0, 16, _zb16, 0)

  def do_pass(p, carry):
    ri = cid * 2 + p
    lo = ri * NQ
    hi = lo + NQ

    def _za(k, c):
      pltpu.sync_copy(zbuf, acc.at[pl.ds(t * RPT0 + k * 8, 8)])
      return c
    lax.fori_loop(0, RPT0 // 8, _za, 0)
    def _zd(k, c):
      pltpu.sync_copy(zbuf16, den.at[pl.ds(t * RPT0 + k * 16, 16)])
      return c
    lax.fori_loop(0, RPT0 // 16, _zd, 0)
    plsc.subcore_barrier()

    def half(hf, hcarry):
      base_e = pl.multiple_of(t * EPT0 + hf * BLKE, 16)
      pltpu.sync_copy(src_hbm.at[pl.ds(base_e, BLKE)], stage_src)
      pltpu.sync_copy(dst_hbm.at[pl.ds(base_e, BLKE)], stage_dst)

      def group(g, cnt):
        dv = stage_dst[pl.ds(g * 16, 16)]
        sv = stage_src[pl.ds(g * 16, 16)]
        m = (dv >= lo) & (dv < hi)
        plsc.store_compressed(csrc.at[pl.ds(cnt, 16)], sv, mask=m)
        plsc.store_compressed(cdst.at[pl.ds(cnt, 16)], dv, mask=m)
        return cnt + plsc.all_reduce_population_count(m)[0]

      cnt = lax.fori_loop(0, BLKE // 16, group, jnp.int32(0))

      csrc[pl.ds(cnt, 16)] = iota16
      cdst[pl.ds(cnt, 16)] = lo + NQ + iota16
      nchunks = (cnt + 15) // 16

      def fire(ci, slot):
        base = pl.multiple_of(ci * 16, 16)
        cs = csrc.at[pl.ds(base, 16)]
        cd = cdst.at[pl.ds(base, 16)]
        pltpu.async_copy(feat_hbm.at[cs], fbuf.at[slot], gsem.at[slot])
        pltpu.async_copy(elrs_hbm.at[cs], esbuf.at[slot], gsem.at[slot])
        pltpu.async_copy(elrd_hbm.at[cd], edbuf.at[slot], gsem.at[slot])

      def wait_gather(slot):
        pltpu.make_async_copy(feat_hbm.at[csrc.at[pl.ds(0, 16)]],
                              fbuf.at[slot], gsem.at[slot]).wait()
        pltpu.make_async_copy(elrs_hbm.at[csrc.at[pl.ds(0, 16)]],
                              esbuf.at[slot], gsem.at[slot]).wait()
        pltpu.make_async_copy(elrd_hbm.at[cdst.at[pl.ds(0, 16)]],
                              edbuf.at[slot], gsem.at[slot]).wait()

      def wait_scatter(slot):
        pltpu.make_async_copy(sbuf.at[slot], acc.at[didx.at[slot]],
                              ssem.at[slot]).wait()
        pltpu.make_async_copy(wbuf.at[slot], den.at[didx.at[slot]],
                              ssem.at[slot]).wait()

      pl.when(nchunks > 0)(lambda: fire(0, 0))
      idxe = ((2 * iota16) & 7)[:, None]
      idxo = ((2 * iota16 + 1) & 7)[:, None]
      dn = lax.GatherDimensionNumbers(offset_dims=(),
                                      collapsed_slice_dims=(0,),
                                      start_index_map=(0,))

      def chunk(ci, c):
        b = lax.rem(ci, 3)
        nb = lax.rem(ci + 1, 3)
        sb = lax.rem(ci, 2)
        pl.when(ci + 1 < nchunks)(lambda: fire(ci + 1, nb))
        pl.when(ci >= 2)(lambda: wait_scatter(sb))
        base = pl.multiple_of(ci * 16, 16)
        dv = cdst[pl.ds(base, 16)]
        didx[sb, :] = dv - lo
        wait_gather(b)
        for e in range(16):
          s = esbuf[b, e, :] + edbuf[b, e, :]
          w = jnp.exp(jnp.where(s > 0, s, s * 0.2))
          wbuf[sb, e, :] = w
          wa = lax.gather(w, idxe, dn, (1,),
                          mode=lax.GatherScatterMode.PROMISE_IN_BOUNDS)
          wo = lax.gather(w, idxo, dn, (1,),
                          mode=lax.GatherScatterMode.PROMISE_IN_BOUNDS)
          for j in range(16):
            ab = fbuf[b, e, pl.ds(j * 32, 32)]
            ae, ao = plsc.unpack(ab, format=plsc.PackFormat.INTERLEAVED)
            sbuf[sb, e, pl.ds(j * 32, 16)] = ae * wa
            sbuf[sb, e, pl.ds(j * 32 + 16, 16)] = ao * wo
        pltpu.async_copy(sbuf.at[sb], acc.at[didx.at[sb]], ssem.at[sb],
                         add=True)
        pltpu.async_copy(wbuf.at[sb], den.at[didx.at[sb]], ssem.at[sb],
                         add=True)
        return c

      lax.fori_loop(0, nchunks, chunk, 0)
      for d in range(2):
        pl.when(nchunks > d)(
            functools.partial(wait_scatter, lax.rem(nchunks - 1 - d + 2, 2)))
      return hcarry

    lax.fori_loop(0, NSUB0, half, 0)
    plsc.subcore_barrier()

    pltpu.sync_copy(acc.at[pl.ds(t * RPT0, RPT0)],
                    num_out.at[ri, pl.ds(t * RPT0, RPT0)])
    pltpu.sync_copy(den.at[pl.ds(t * RPT0, RPT0)],
                    den_out.at[ri, pl.ds(t * RPT0, RPT0)])
    plsc.subcore_barrier()
    return carry

  lax.fori_loop(0, 2, do_pass, 0)


def _edge1_body(feat1_hbm, e1s_hbm, e1d_hbm, src_hbm, dst_hbm,
                num_out, den_out,
                stage_src, stage_dst, didx,
                fbuf, esbuf, edbuf, wbuf, zbuf48, zbufd,
                acc, den, gsem, ssem):
  cid = lax.axis_index("c")
  t = lax.axis_index("s")
  w32 = t * 2 + cid
  iota16 = lax.iota(I32, 16)
  zero16 = jnp.zeros((16,), F32)

  def _zb(r, c):
    for j in range(3):
      zbuf48[r, pl.ds(j * 16, 16)] = zero16
    zbufd[r, :] = zero16
    return c
  lax.fori_loop(0, RPT1, _zb, 0)
  pltpu.sync_copy(zbuf48, acc.at[pl.ds(t * RPT1, RPT1)])
  pltpu.sync_copy(zbufd, den.at[pl.ds(t * RPT1, RPT1)])

  base_e = pl.multiple_of(w32 * EPT1, 16)
  pltpu.sync_copy(src_hbm.at[pl.ds(base_e, EPT1)], stage_src.at[pl.ds(0, EPT1)])
  pltpu.sync_copy(dst_hbm.at[pl.ds(base_e, EPT1)], stage_dst.at[pl.ds(0, EPT1)])
  stage_src[pl.ds(EPT1, 16)] = iota16
  stage_dst[pl.ds(EPT1, 16)] = N + iota16
  plsc.subcore_barrier()

  def fire(q, slot):
    qb = pl.multiple_of(q * 32, 32)
    cs = stage_src.at[pl.ds(qb, 32)]
    cd = stage_dst.at[pl.ds(qb, 32)]
    pltpu.async_copy(feat1_hbm.at[cs], fbuf.at[slot], gsem.at[slot])
    pltpu.async_copy(e1s_hbm.at[cs], esbuf.at[slot], gsem.at[slot])
    pltpu.async_copy(e1d_hbm.at[cd], edbuf.at[slot], gsem.at[slot])

  def wait_gather(slot):
    pltpu.make_async_copy(feat1_hbm.at[stage_src.at[pl.ds(0, 32)]],
                          fbuf.at[slot], gsem.at[slot]).wait()
    pltpu.make_async_copy(e1s_hbm.at[stage_src.at[pl.ds(0, 32)]],
                          esbuf.at[slot], gsem.at[slot]).wait()
    pltpu.make_async_copy(e1d_hbm.at[stage_dst.at[pl.ds(0, 32)]],
                          edbuf.at[slot], gsem.at[slot]).wait()

  def wait_scatter(slot):
    pltpu.make_async_copy(fbuf.at[slot], acc.at[didx.at[slot]],
                          ssem.at[slot]).wait()
    pltpu.make_async_copy(wbuf.at[slot], den.at[didx.at[slot]],
                          ssem.at[slot]).wait()

  fire(0, 0)

  def chunk(q, c):
    b = lax.rem(q, 3)
    nb = lax.rem(q + 1, 3)
    pl.when(q + 1 < NCH1)(lambda: fire(q + 1, nb))
    pl.when(q >= 3)(lambda: wait_scatter(b))
    qb = pl.multiple_of(q * 32, 32)
    didx[b, pl.ds(0, 16)] = stage_dst[pl.ds(qb, 16)]
    didx[b, pl.ds(16, 16)] = stage_dst[pl.ds(qb + 16, 16)]
    wait_gather(b)
    for e in range(32):
      s = esbuf[b, e, :] + edbuf[b, e, :]
      wbuf[b, e, :] = jnp.exp(jnp.where(s > 0, s, s * 0.2))
    for e in range(32):
      w = wbuf[b, e, :]
      for j in range(3):
        sl = pl.ds(j * 16, 16)
        fbuf[b, e, sl] = fbuf[b, e, sl] * w
    pltpu.async_copy(fbuf.at[b], acc.at[didx.at[b]], ssem.at[b], add=True)
    pltpu.async_copy(wbuf.at[b], den.at[didx.at[b]], ssem.at[b], add=True)
    return c

  lax.fori_loop(0, NCH1, chunk, 0)
  for d in range(3):
    wait_scatter((NCH1 - 1 - d) % 3)
  plsc.subcore_barrier()
  pltpu.sync_copy(acc.at[pl.ds(t * RPT1, RPT1)],
                  num_out.at[cid, pl.ds(t * RPT1, RPT1)])
  pltpu.sync_copy(den.at[pl.ds(t * RPT1, RPT1)],
                  den_out.at[cid, pl.ds(t * RPT1, RPT1)])


_sc_mesh = plsc.VectorSubcoreMesh(core_axis_name="c", subcore_axis_name="s")
_sc_params = pltpu.CompilerParams(
    needs_layout_passes=False, use_tc_tiling_on_sc=False)

_edge0 = pl.kernel(
    _edge0_body,
    compiler_params=_sc_params,
    out_type=(jax.ShapeDtypeStruct((NRANGE, ACCR, 512), F32),
              jax.ShapeDtypeStruct((NRANGE, ACCR, 16), F32)),
    mesh=_sc_mesh,
    scratch_types=[
        pltpu.VMEM((BLKE,), I32),
        pltpu.VMEM((BLKE,), I32),
        pltpu.VMEM((BLKE + 16,), I32),
        pltpu.VMEM((BLKE + 16,), I32),
        pltpu.VMEM((2, 16), I32),
        pltpu.VMEM((3, 16, 512), jnp.bfloat16),
        pltpu.VMEM((2, 16, 512), F32),
        pltpu.VMEM((3, 16, 16), F32),
        pltpu.VMEM((3, 16, 16), F32),
        pltpu.VMEM((2, 16, 16), F32),
        pltpu.VMEM((8, 512), F32),
        pltpu.VMEM((16, 16), F32),
        pltpu.VMEM_SHARED((ACCR, 512), F32),
        pltpu.VMEM_SHARED((ACCR, 16), F32),
        pltpu.SemaphoreType.DMA((3,)),
        pltpu.SemaphoreType.DMA((3,)),
    ])

_edge1 = pl.kernel(
    _edge1_body,
    compiler_params=_sc_params,
    out_type=(jax.ShapeDtypeStruct((2, ACC1R, 48), F32),
              jax.ShapeDtypeStruct((2, ACC1R, 16), F32)),
    mesh=_sc_mesh,
    scratch_types=[
        pltpu.VMEM((EPT1 + 16,), I32),
        pltpu.VMEM((EPT1 + 16,), I32),
        pltpu.VMEM((3, 32), I32),
        pltpu.VMEM((3, 32, 48), F32),
        pltpu.VMEM((3, 32, 16), F32),
        pltpu.VMEM((3, 32, 16), F32),
        pltpu.VMEM((3, 32, 16), F32),
        pltpu.VMEM((RPT1, 48), F32),
        pltpu.VMEM((RPT1, 16), F32),
        pltpu.VMEM_SHARED((ACC1R, 48), F32),
        pltpu.VMEM_SHARED((ACC1R, 16), F32),
        pltpu.SemaphoreType.DMA((3,)),
        pltpu.SemaphoreType.DMA((3,)),
    ])


def _dense0_body(x_ref, w_ref, a_ref, b_ref, feat_ref, es_ref, ed_ref):
  f = jnp.dot(x_ref[...], w_ref[...], preferred_element_type=F32)
  feat_ref[...] = f.astype(jnp.bfloat16)
  es_ref[...] = jnp.dot(f, a_ref[...], preferred_element_type=F32)
  ed_ref[...] = jnp.dot(f, b_ref[...], preferred_element_type=F32)


_dense0 = pl.pallas_call(
    _dense0_body,
    grid=(5,),
    in_specs=[
        pl.BlockSpec((2000, 128), lambda i: (i, 0)),
        pl.BlockSpec((128, 512), lambda i: (0, 0)),
        pl.BlockSpec((512, 16), lambda i: (0, 0)),
        pl.BlockSpec((512, 16), lambda i: (0, 0)),
    ],
    out_specs=[
        pl.BlockSpec((2000, 512), lambda i: (i, 0)),
        pl.BlockSpec((2000, 16), lambda i: (i, 0)),
        pl.BlockSpec((2000, 16), lambda i: (i, 0)),
    ],
    out_shape=[
        jax.ShapeDtypeStruct((N, 512), jnp.bfloat16),
        jax.ShapeDtypeStruct((N, 16), F32),
        jax.ShapeDtypeStruct((N, 16), F32),
    ])


def _node0_body(num_ref, den_ref, b0_ref, s_ref, w1_ref, wres_ref, a1_ref,
                b1a_ref, feat1_ref, res_ref, e1s_ref, e1d_ref):
  nb = num_ref[...]
  db = den_ref[...]
  dbig = jnp.dot(db, s_ref[...], preferred_element_type=F32)
  pre = nb / (dbig + 1e-9) + b0_ref[...]
  h = jnp.where(pre > 0, pre, jnp.exp(pre) - 1.0)
  feat1_ref[...] = jnp.dot(h, w1_ref[...], preferred_element_type=F32)
  res_ref[...] = jnp.dot(h, wres_ref[...], preferred_element_type=F32)
  e1s_ref[...] = jnp.dot(h, a1_ref[...], preferred_element_type=F32)
  e1d_ref[...] = jnp.dot(h, b1a_ref[...], preferred_element_type=F32)


_node0 = pl.pallas_call(
    _node0_body,
    grid=(10,),
    in_specs=[
        pl.BlockSpec((1000, 512), lambda i: (i, 0)),
        pl.BlockSpec((1000, 16), lambda i: (i, 0)),
        pl.BlockSpec((1, 512), lambda i: (0, 0)),
        pl.BlockSpec((16, 512), lambda i: (0, 0)),
        pl.BlockSpec((512, 48), lambda i: (0, 0)),
        pl.BlockSpec((512, 48), lambda i: (0, 0)),
        pl.BlockSpec((512, 16), lambda i: (0, 0)),
        pl.BlockSpec((512, 16), lambda i: (0, 0)),
    ],
    out_specs=[
        pl.BlockSpec((1000, 48), lambda i: (i, 0)),
        pl.BlockSpec((1000, 48), lambda i: (i, 0)),
        pl.BlockSpec((1000, 16), lambda i: (i, 0)),
        pl.BlockSpec((1000, 16), lambda i: (i, 0)),
    ],
    out_shape=[
        jax.ShapeDtypeStruct((N, 48), F32),
        jax.ShapeDtypeStruct((N, 48), F32),
        jax.ShapeDtypeStruct((N, 16), F32),
        jax.ShapeDtypeStruct((N, 16), F32),
    ])


def _final_body(n1_ref, d1_ref, res_ref, b1_ref, s1_ref, out_ref):
  ns = n1_ref[0] + n1_ref[1]
  dsum = d1_ref[0] + d1_ref[1]
  dbig = jnp.dot(dsum, s1_ref[...], preferred_element_type=F32)
  out_ref[...] = ns / (dbig + 1e-9) + res_ref[...] + b1_ref[...]


_final = pl.pallas_call(
    _final_body,
    grid=(10,),
    in_specs=[
        pl.BlockSpec((2, 1000, 48), lambda i: (0, i, 0)),
        pl.BlockSpec((2, 1000, 16), lambda i: (0, i, 0)),
        pl.BlockSpec((1000, 48), lambda i: (i, 0)),
        pl.BlockSpec((1, 48), lambda i: (0, 0)),
        pl.BlockSpec((16, 48), lambda i: (0, 0)),
    ],
    out_specs=pl.BlockSpec((1000, 48), lambda i: (i, 0)),
    out_shape=jax.ShapeDtypeStruct((N, 48), F32))


def kernel(x, edge_index0, edge_index1, W0, al0, ar0, b0, W1, al1, ar1, b1,
           Wres):
  W0_il = W0.reshape(128, 8, 64).transpose(0, 2, 1).reshape(128, 512)
  al_flat = al0.T.reshape(512)
  ar_flat = ar0.T.reshape(512)
  eyea = jnp.asarray(_EYEA)
  Ail = al_flat[:, None] * eyea
  Bil = ar_flat[:, None] * eyea
  def shuf(v):
    rest = v.shape[1:]
    return v.reshape(16, 16, 2, *rest).swapaxes(1, 2).reshape(512, *rest)

  b0_il = shuf(b0.reshape(8, 64).T.reshape(512))[None, :]
  S_il = jnp.asarray(_SIL)
  W1p = shuf(jnp.pad(W1, ((0, 0), (0, 8))).reshape(8, 64, 48).transpose(1, 0, 2).reshape(512, 48))
  Wresp = shuf(jnp.pad(Wres, ((0, 0), (0, 8))).reshape(8, 64, 48).transpose(1, 0, 2).reshape(512, 48))
  a1c = shuf((W1 @ al1[0]).reshape(8, 64).T.reshape(512))
  b1c = shuf((W1 @ ar1[0]).reshape(8, 64).T.reshape(512))
  A1 = jnp.tile(a1c[:, None], (1, 16)).astype(F32)
  B1 = jnp.tile(b1c[:, None], (1, 16)).astype(F32)
  b1p = jnp.pad(b1, (0, 8))[None, :]
  S1 = jnp.asarray(_S1)

  src0, dst0 = edge_index0[0], edge_index0[1]
  src1, dst1 = edge_index1[0], edge_index1[1]

  feat, elrS, elrD = _dense0(x, W0_il, Ail, Bil)
  elrD_p = jnp.pad(elrD, ((0, NPAD0 - N), (0, 0)))
  num0, den0 = _edge0(feat, elrS, elrD_p, src0, dst0)
  num0f = num0[:, :NQ, :].reshape(NRANGE * NQ, 512)[:N]
  den0f = den0[:, :NQ, :].reshape(NRANGE * NQ, 16)[:N]
  feat1, res, e1s, e1d = _node0(num0f, den0f, b0_il, S_il, W1p, Wresp, A1, B1)
  e1d_p = jnp.pad(e1d, ((0, 16), (0, 0)))
  num1, den1 = _edge1(feat1, e1s, e1d_p, src1, dst1)
  out48 = _final(num1, den1, res, b1p, S1)
  return out48[:, :40]


# speedup vs baseline: 2.3390x; 2.3390x over previous
import functools

import jax
import jax.numpy as jnp
import numpy as np
from jax import lax
from jax.experimental import pallas as pl
from jax.experimental.pallas import tpu as pltpu
from jax.experimental.pallas import tpu_sc as plsc

N = 10000
E = 320000
F32 = jnp.float32
I32 = jnp.int32

NRANGE = 4
NQ = 2500
ACCR = 2560
RPT0 = 160
EPT0 = E // 16
BLKE = 2000
NSUB0 = EPT0 // BLKE
NPAD0 = NRANGE * NQ + 16

ACC1R = 10112
RPT1 = 632
EPT1 = E // 32
NBLK1 = EPT1 // BLKE
NCH1 = (EPT1 + 16) // 32

_Q = np.arange(512)
_HH = _Q % 8
_EYEA = np.zeros((512, 16), np.float32)
_EYEA[_Q, _HH] = 1.0
_EYEA[_Q, _HH + 8] = 1.0
_SIL = np.zeros((16, 512), np.float32)
_SIL[_HH, _Q] = 1.0
_S1 = np.zeros((16, 48), np.float32)
_S1[0, :] = 1.0


def _edge0_body(feat_hbm, elrs_hbm, elrd_hbm, src_hbm, dst_hbm,
                num_out, den_out,
                stage_src, stage_dst, csrc, cdst, didx,
                fbuf, esbuf, edbuf, wbuf, zbuf, zbuf16,
                acc, den, gsem, ssem):
  cid = lax.axis_index("c")
  t = lax.axis_index("s")
  iota16 = lax.iota(I32, 16)
  zero16 = jnp.zeros((16,), F32)

  def _zb(r, c):
    for j in range(32):
      zbuf[r, pl.ds(j * 16, 16)] = zero16
    return c
  lax.fori_loop(0, 8, _zb, 0)

  def _zb16(r, c):
    zbuf16[r, :] = zero16
    return c
  lax.fori_loop(0, 16, _zb16, 0)

  def do_pass(p, carry):
    ri = cid * 2 + p
    lo = ri * NQ
    hi = lo + NQ

    def _za(k, c):
      pltpu.sync_copy(zbuf, acc.at[pl.ds(t * RPT0 + k * 8, 8)])
      return c
    lax.fori_loop(0, RPT0 // 8, _za, 0)
    def _zd(k, c):
      pltpu.sync_copy(zbuf16, den.at[pl.ds(t * RPT0 + k * 16, 16)])
      return c
    lax.fori_loop(0, RPT0 // 16, _zd, 0)
    plsc.subcore_barrier()

    def half(hf, hcarry):
      base_e = pl.multiple_of(t * EPT0 + hf * BLKE, 16)
      pltpu.sync_copy(src_hbm.at[pl.ds(base_e, BLKE)], stage_src)
      pltpu.sync_copy(dst_hbm.at[pl.ds(base_e, BLKE)], stage_dst)

      def group(g, cnt):
        dv = stage_dst[pl.ds(g * 16, 16)]
        sv = stage_src[pl.ds(g * 16, 16)]
        m = (dv >= lo) & (dv < hi)
        plsc.store_compressed(csrc.at[pl.ds(cnt, 16)], sv, mask=m)
        plsc.store_compressed(cdst.at[pl.ds(cnt, 16)], dv, mask=m)
        return cnt + plsc.all_reduce_population_count(m)[0]

      cnt = lax.fori_loop(0, BLKE // 16, group, jnp.int32(0))

      csrc[pl.ds(cnt, 16)] = iota16
      cdst[pl.ds(cnt, 16)] = lo + NQ + iota16
      nchunks = (cnt + 15) // 16

      def fire(ci, slot):
        base = pl.multiple_of(ci * 16, 16)
        cs = csrc.at[pl.ds(base, 16)]
        cd = cdst.at[pl.ds(base, 16)]
        pltpu.async_copy(feat_hbm.at[cs], fbuf.at[slot], gsem.at[slot])
        pltpu.async_copy(elrs_hbm.at[cs], esbuf.at[slot], gsem.at[slot])
        pltpu.async_copy(elrd_hbm.at[cd], edbuf.at[slot], gsem.at[slot])

      def wait_gather(slot):
        pltpu.make_async_copy(feat_hbm.at[csrc.at[pl.ds(0, 16)]],
                              fbuf.at[slot], gsem.at[slot]).wait()
        pltpu.make_async_copy(elrs_hbm.at[csrc.at[pl.ds(0, 16)]],
                              esbuf.at[slot], gsem.at[slot]).wait()
        pltpu.make_async_copy(elrd_hbm.at[cdst.at[pl.ds(0, 16)]],
                              edbuf.at[slot], gsem.at[slot]).wait()

      def wait_scatter(slot):
        pltpu.make_async_copy(fbuf.at[slot], acc.at[didx.at[slot]],
                              ssem.at[slot]).wait()
        pltpu.make_async_copy(wbuf.at[slot], den.at[didx.at[slot]],
                              ssem.at[slot]).wait()

      pl.when(nchunks > 0)(lambda: fire(0, 0))

      def chunk(ci, c):
        b = lax.rem(ci, 3)
        nb = lax.rem(ci + 1, 3)
        pl.when(ci + 1 < nchunks)(lambda: fire(ci + 1, nb))
        pl.when(ci >= 3)(lambda: wait_scatter(b))
        base = pl.multiple_of(ci * 16, 16)
        dv = cdst[pl.ds(base, 16)]
        didx[b, :] = dv - lo
        wait_gather(b)
        for e in range(16):
          s = esbuf[b, e, :] + edbuf[b, e, :]
          wbuf[b, e, :] = jnp.exp(jnp.where(s > 0, s, s * 0.2))
        for e in range(16):
          w = wbuf[b, e, :]
          for j in range(32):
            sl = pl.ds(j * 16, 16)
            fbuf[b, e, sl] = fbuf[b, e, sl] * w
        pltpu.async_copy(fbuf.at[b], acc.at[didx.at[b]], ssem.at[b],
                         add=True)
        pltpu.async_copy(wbuf.at[b], den.at[didx.at[b]], ssem.at[b],
                         add=True)
        return c

      lax.fori_loop(0, nchunks, chunk, 0)
      for d in range(3):
        pl.when(nchunks > d)(
            functools.partial(wait_scatter, lax.rem(nchunks - 1 - d + 3, 3)))
      return hcarry

    lax.fori_loop(0, NSUB0, half, 0)
    plsc.subcore_barrier()

    pltpu.sync_copy(acc.at[pl.ds(t * RPT0, RPT0)],
                    num_out.at[ri, pl.ds(t * RPT0, RPT0)])
    pltpu.sync_copy(den.at[pl.ds(t * RPT0, RPT0)],
                    den_out.at[ri, pl.ds(t * RPT0, RPT0)])
    plsc.subcore_barrier()
    return carry

  lax.fori_loop(0, 2, do_pass, 0)


def _edge1_body(feat1_hbm, e1s_hbm, e1d_hbm, src_hbm, dst_hbm,
                num_out, den_out,
                stage_src, stage_dst, didx,
                fbuf, esbuf, edbuf, wbuf, zbuf48, zbufd,
                acc, den, gsem, ssem):
  cid = lax.axis_index("c")
  t = lax.axis_index("s")
  w32 = t * 2 + cid
  iota16 = lax.iota(I32, 16)
  zero16 = jnp.zeros((16,), F32)

  def _zb(r, c):
    for j in range(3):
      zbuf48[r, pl.ds(j * 16, 16)] = zero16
    zbufd[r, :] = zero16
    return c
  lax.fori_loop(0, RPT1, _zb, 0)
  pltpu.sync_copy(zbuf48, acc.at[pl.ds(t * RPT1, RPT1)])
  pltpu.sync_copy(zbufd, den.at[pl.ds(t * RPT1, RPT1)])

  base_e = pl.multiple_of(w32 * EPT1, 16)
  pltpu.sync_copy(src_hbm.at[pl.ds(base_e, EPT1)], stage_src.at[pl.ds(0, EPT1)])
  pltpu.sync_copy(dst_hbm.at[pl.ds(base_e, EPT1)], stage_dst.at[pl.ds(0, EPT1)])
  stage_src[pl.ds(EPT1, 16)] = iota16
  stage_dst[pl.ds(EPT1, 16)] = N + iota16
  plsc.subcore_barrier()

  def fire(q, slot):
    qb = pl.multiple_of(q * 32, 32)
    cs = stage_src.at[pl.ds(qb, 32)]
    cd = stage_dst.at[pl.ds(qb, 32)]
    pltpu.async_copy(feat1_hbm.at[cs], fbuf.at[slot], gsem.at[slot])
    pltpu.async_copy(e1s_hbm.at[cs], esbuf.at[slot], gsem.at[slot])
    pltpu.async_copy(e1d_hbm.at[cd], edbuf.at[slot], gsem.at[slot])

  def wait_gather(slot):
    pltpu.make_async_copy(feat1_hbm.at[stage_src.at[pl.ds(0, 32)]],
                          fbuf.at[slot], gsem.at[slot]).wait()
    pltpu.make_async_copy(e1s_hbm.at[stage_src.at[pl.ds(0, 32)]],
                          esbuf.at[slot], gsem.at[slot]).wait()
    pltpu.make_async_copy(e1d_hbm.at[stage_dst.at[pl.ds(0, 32)]],
                          edbuf.at[slot], gsem.at[slot]).wait()

  def wait_scatter(slot):
    pltpu.make_async_copy(fbuf.at[slot], acc.at[didx.at[slot]],
                          ssem.at[slot]).wait()
    pltpu.make_async_copy(wbuf.at[slot], den.at[didx.at[slot]],
                          ssem.at[slot]).wait()

  fire(0, 0)

  def chunk(q, c):
    b = lax.rem(q, 3)
    nb = lax.rem(q + 1, 3)
    pl.when(q + 1 < NCH1)(lambda: fire(q + 1, nb))
    pl.when(q >= 3)(lambda: wait_scatter(b))
    qb = pl.multiple_of(q * 32, 32)
    didx[b, pl.ds(0, 16)] = stage_dst[pl.ds(qb, 16)]
    didx[b, pl.ds(16, 16)] = stage_dst[pl.ds(qb + 16, 16)]
    wait_gather(b)
    for e in range(32):
      s = esbuf[b, e, :] + edbuf[b, e, :]
      wbuf[b, e, :] = jnp.exp(jnp.where(s > 0, s, s * 0.2))
    for e in range(32):
      w = wbuf[b, e, :]
      for j in range(3):
        sl = pl.ds(j * 16, 16)
        fbuf[b, e, sl] = fbuf[b, e, sl] * w
    pltpu.async_copy(fbuf.at[b], acc.at[didx.at[b]], ssem.at[b], add=True)
    pltpu.async_copy(wbuf.at[b], den.at[didx.at[b]], ssem.at[b], add=True)
    return c

  lax.fori_loop(0, NCH1, chunk, 0)
  for d in range(3):
    wait_scatter((NCH1 - 1 - d) % 3)
  plsc.subcore_barrier()
  pltpu.sync_copy(acc.at[pl.ds(t * RPT1, RPT1)],
                  num_out.at[cid, pl.ds(t * RPT1, RPT1)])
  pltpu.sync_copy(den.at[pl.ds(t * RPT1, RPT1)],
                  den_out.at[cid, pl.ds(t * RPT1, RPT1)])


_sc_mesh = plsc.VectorSubcoreMesh(core_axis_name="c", subcore_axis_name="s")
_sc_params = pltpu.CompilerParams(
    needs_layout_passes=False, use_tc_tiling_on_sc=False)

_edge0 = pl.kernel(
    _edge0_body,
    compiler_params=_sc_params,
    out_type=(jax.ShapeDtypeStruct((NRANGE, ACCR, 512), F32),
              jax.ShapeDtypeStruct((NRANGE, ACCR, 16), F32)),
    mesh=_sc_mesh,
    scratch_types=[
        pltpu.VMEM((BLKE,), I32),
        pltpu.VMEM((BLKE,), I32),
        pltpu.VMEM((BLKE + 16,), I32),
        pltpu.VMEM((BLKE + 16,), I32),
        pltpu.VMEM((3, 16), I32),
        pltpu.VMEM((3, 16, 512), F32),
        pltpu.VMEM((3, 16, 16), F32),
        pltpu.VMEM((3, 16, 16), F32),
        pltpu.VMEM((3, 16, 16), F32),
        pltpu.VMEM((8, 512), F32),
        pltpu.VMEM((16, 16), F32),
        pltpu.VMEM_SHARED((ACCR, 512), F32),
        pltpu.VMEM_SHARED((ACCR, 16), F32),
        pltpu.SemaphoreType.DMA((3,)),
        pltpu.SemaphoreType.DMA((3,)),
    ])

_edge1 = pl.kernel(
    _edge1_body,
    compiler_params=_sc_params,
    out_type=(jax.ShapeDtypeStruct((2, ACC1R, 48), F32),
              jax.ShapeDtypeStruct((2, ACC1R, 16), F32)),
    mesh=_sc_mesh,
    scratch_types=[
        pltpu.VMEM((EPT1 + 16,), I32),
        pltpu.VMEM((EPT1 + 16,), I32),
        pltpu.VMEM((3, 32), I32),
        pltpu.VMEM((3, 32, 48), F32),
        pltpu.VMEM((3, 32, 16), F32),
        pltpu.VMEM((3, 32, 16), F32),
        pltpu.VMEM((3, 32, 16), F32),
        pltpu.VMEM((RPT1, 48), F32),
        pltpu.VMEM((RPT1, 16), F32),
        pltpu.VMEM_SHARED((ACC1R, 48), F32),
        pltpu.VMEM_SHARED((ACC1R, 16), F32),
        pltpu.SemaphoreType.DMA((3,)),
        pltpu.SemaphoreType.DMA((3,)),
    ])


def _dense0_body(x_ref, w_ref, a_ref, b_ref, feat_ref, es_ref, ed_ref):
  f = jnp.dot(x_ref[...], w_ref[...], preferred_element_type=F32)
  feat_ref[...] = f
  es_ref[...] = jnp.dot(f, a_ref[...], preferred_element_type=F32)
  ed_ref[...] = jnp.dot(f, b_ref[...], preferred_element_type=F32)


_dense0 = pl.pallas_call(
    _dense0_body,
    grid=(10,),
    in_specs=[
        pl.BlockSpec((1000, 128), lambda i: (i, 0)),
        pl.BlockSpec((128, 512), lambda i: (0, 0)),
        pl.BlockSpec((512, 16), lambda i: (0, 0)),
        pl.BlockSpec((512, 16), lambda i: (0, 0)),
    ],
    out_specs=[
        pl.BlockSpec((1000, 512), lambda i: (i, 0)),
        pl.BlockSpec((1000, 16), lambda i: (i, 0)),
        pl.BlockSpec((1000, 16), lambda i: (i, 0)),
    ],
    out_shape=[
        jax.ShapeDtypeStruct((N, 512), F32),
        jax.ShapeDtypeStruct((N, 16), F32),
        jax.ShapeDtypeStruct((N, 16), F32),
    ])


def _node0_body(num_ref, den_ref, b0_ref, s_ref, w1_ref, wres_ref, a1_ref,
                b1a_ref, feat1_ref, res_ref, e1s_ref, e1d_ref):
  nb = num_ref[...]
  db = den_ref[...]
  dbig = jnp.dot(db, s_ref[...], preferred_element_type=F32)
  pre = nb / (dbig + 1e-9) + b0_ref[...]
  h = jnp.where(pre > 0, pre, jnp.exp(pre) - 1.0)
  feat1_ref[...] = jnp.dot(h, w1_ref[...], preferred_element_type=F32)
  res_ref[...] = jnp.dot(h, wres_ref[...], preferred_element_type=F32)
  e1s_ref[...] = jnp.dot(h, a1_ref[...], preferred_element_type=F32)
  e1d_ref[...] = jnp.dot(h, b1a_ref[...], preferred_element_type=F32)


_node0 = pl.pallas_call(
    _node0_body,
    grid=(10,),
    in_specs=[
        pl.BlockSpec((1000, 512), lambda i: (i, 0)),
        pl.BlockSpec((1000, 16), lambda i: (i, 0)),
        pl.BlockSpec((1, 512), lambda i: (0, 0)),
        pl.BlockSpec((16, 512), lambda i: (0, 0)),
        pl.BlockSpec((512, 48), lambda i: (0, 0)),
        pl.BlockSpec((512, 48), lambda i: (0, 0)),
        pl.BlockSpec((512, 16), lambda i: (0, 0)),
        pl.BlockSpec((512, 16), lambda i: (0, 0)),
    ],
    out_specs=[
        pl.BlockSpec((1000, 48), lambda i: (i, 0)),
        pl.BlockSpec((1000, 48), lambda i: (i, 0)),
        pl.BlockSpec((1000, 16), lambda i: (i, 0)),
        pl.BlockSpec((1000, 16), lambda i: (i, 0)),
    ],
    out_shape=[
        jax.ShapeDtypeStruct((N, 48), F32),
        jax.ShapeDtypeStruct((N, 48), F32),
        jax.ShapeDtypeStruct((N, 16), F32),
        jax.ShapeDtypeStruct((N, 16), F32),
    ])


def _final_body(n1_ref, d1_ref, res_ref, b1_ref, s1_ref, out_ref):
  ns = n1_ref[0] + n1_ref[1]
  dsum = d1_ref[0] + d1_ref[1]
  dbig = jnp.dot(dsum, s1_ref[...], preferred_element_type=F32)
  out_ref[...] = ns / (dbig + 1e-9) + res_ref[...] + b1_ref[...]


_final = pl.pallas_call(
    _final_body,
    grid=(10,),
    in_specs=[
        pl.BlockSpec((2, 1000, 48), lambda i: (0, i, 0)),
        pl.BlockSpec((2, 1000, 16), lambda i: (0, i, 0)),
        pl.BlockSpec((1000, 48), lambda i: (i, 0)),
        pl.BlockSpec((1, 48), lambda i: (0, 0)),
        pl.BlockSpec((16, 48), lambda i: (0, 0)),
    ],
    out_specs=pl.BlockSpec((1000, 48), lambda i: (i, 0)),
    out_shape=jax.ShapeDtypeStruct((N, 48), F32))


def kernel(x, edge_index0, edge_index1, W0, al0, ar0, b0, W1, al1, ar1, b1,
           Wres):
  W0_il = W0.reshape(128, 8, 64).transpose(0, 2, 1).reshape(128, 512)
  al_flat = al0.T.reshape(512)
  ar_flat = ar0.T.reshape(512)
  eyea = jnp.asarray(_EYEA)
  Ail = al_flat[:, None] * eyea
  Bil = ar_flat[:, None] * eyea
  b0_il = b0.reshape(8, 64).T.reshape(512)[None, :]
  S_il = jnp.asarray(_SIL)
  W1p = jnp.pad(W1, ((0, 0), (0, 8))).reshape(8, 64, 48).transpose(1, 0, 2).reshape(512, 48)
  Wresp = jnp.pad(Wres, ((0, 0), (0, 8))).reshape(8, 64, 48).transpose(1, 0, 2).reshape(512, 48)
  a1c = (W1 @ al1[0]).reshape(8, 64).T.reshape(512)
  b1c = (W1 @ ar1[0]).reshape(8, 64).T.reshape(512)
  A1 = jnp.tile(a1c[:, None], (1, 16)).astype(F32)
  B1 = jnp.tile(b1c[:, None], (1, 16)).astype(F32)
  b1p = jnp.pad(b1, (0, 8))[None, :]
  S1 = jnp.asarray(_S1)

  src0, dst0 = edge_index0[0], edge_index0[1]
  src1, dst1 = edge_index1[0], edge_index1[1]

  feat, elrS, elrD = _dense0(x, W0_il, Ail, Bil)
  elrD_p = jnp.pad(elrD, ((0, NPAD0 - N), (0, 0)))
  num0, den0 = _edge0(feat, elrS, elrD_p, src0, dst0)
  num0f = num0[:, :NQ, :].reshape(NRANGE * NQ, 512)[:N]
  den0f = den0[:, :NQ, :].reshape(NRANGE * NQ, 16)[:N]
  feat1, res, e1s, e1d = _node0(num0f, den0f, b0_il, S_il, W1p, Wresp, A1, B1)
  e1d_p = jnp.pad(e1d, ((0, 16), (0, 0)))
  num1, den1 = _edge1(feat1, e1s, e1d_p, src1, dst1)
  out48 = _final(num1, den1, res, b1p, S1)
  return out48[:, :40]


# speedup vs baseline: 2.4585x; 1.0511x over previous
import functools

import jax
import jax.numpy as jnp
import numpy as np
from jax import lax
from jax.experimental import pallas as pl
from jax.experimental.pallas import tpu as pltpu
from jax.experimental.pallas import tpu_sc as plsc

N = 10000
E = 320000
F32 = jnp.float32
I32 = jnp.int32

NRANGE = 4
NQ = 2500
ACCR = 2560
RPT0 = 160
EPT0 = E // 16
BLKE = 2000
NSUB0 = EPT0 // BLKE
NPAD0 = NRANGE * NQ + 16

ACC1R = 10112
RPT1 = 632
EPT1 = E // 32
NBLK1 = EPT1 // BLKE
NCH1 = (EPT1 + 48) // 64

_Q = np.arange(512)
_HH = _Q % 8
_EYEA = np.zeros((512, 16), np.float32)
_EYEA[_Q, _HH] = 1.0
_EYEA[_Q, _HH + 8] = 1.0
_SIL = np.zeros((16, 512), np.float32)
_SIL[_HH, _Q] = 1.0
_S1 = np.zeros((16, 48), np.float32)
_S1[0, :] = 1.0


def _edge0_body(feat_hbm, elrs_hbm, elrd_hbm, src_hbm, dst_hbm,
                num_out, den_out,
                stage_src, stage_dst, csrc, cdst, didx,
                fbuf, esbuf, edbuf, wbuf, zbuf, zbuf16,
                acc, den, gsem, ssem):
  cid = lax.axis_index("c")
  t = lax.axis_index("s")
  iota16 = lax.iota(I32, 16)
  zero16 = jnp.zeros((16,), F32)

  def _zb(r, c):
    for j in range(32):
      zbuf[r, pl.ds(j * 16, 16)] = zero16
    return c
  lax.fori_loop(0, 8, _zb, 0)

  def _zb16(r, c):
    zbuf16[r, :] = zero16
    return c
  lax.fori_loop(0, 16, _zb16, 0)

  def do_pass(p, carry):
    ri = cid * 2 + p
    lo = ri * NQ
    hi = lo + NQ

    def _za(k, c):
      pltpu.sync_copy(zbuf, acc.at[pl.ds(t * RPT0 + k * 8, 8)])
      return c
    lax.fori_loop(0, RPT0 // 8, _za, 0)
    def _zd(k, c):
      pltpu.sync_copy(zbuf16, den.at[pl.ds(t * RPT0 + k * 16, 16)])
      return c
    lax.fori_loop(0, RPT0 // 16, _zd, 0)
    plsc.subcore_barrier()

    def half(hf, hcarry):
      base_e = pl.multiple_of(t * EPT0 + hf * BLKE, 16)
      pltpu.sync_copy(src_hbm.at[pl.ds(base_e, BLKE)], stage_src)
      pltpu.sync_copy(dst_hbm.at[pl.ds(base_e, BLKE)], stage_dst)

      def group(g, cnt):
        dv = stage_dst[pl.ds(g * 16, 16)]
        sv = stage_src[pl.ds(g * 16, 16)]
        m = (dv >= lo) & (dv < hi)
        plsc.store_compressed(csrc.at[pl.ds(cnt, 16)], sv, mask=m)
        plsc.store_compressed(cdst.at[pl.ds(cnt, 16)], dv, mask=m)
        return cnt + plsc.all_reduce_population_count(m)[0]

      cnt = lax.fori_loop(0, BLKE // 16, group, jnp.int32(0))

      csrc[pl.ds(cnt, 16)] = iota16
      cdst[pl.ds(cnt, 16)] = lo + NQ + iota16
      nchunks = (cnt + 15) // 16

      def fire(ci, slot):
        base = pl.multiple_of(ci * 16, 16)
        cs = csrc.at[pl.ds(base, 16)]
        cd = cdst.at[pl.ds(base, 16)]
        pltpu.async_copy(feat_hbm.at[cs], fbuf.at[slot], gsem.at[slot])
        pltpu.async_copy(elrs_hbm.at[cs], esbuf.at[slot], gsem.at[slot])
        pltpu.async_copy(elrd_hbm.at[cd], edbuf.at[slot], gsem.at[slot])

      def wait_gather(slot):
        pltpu.make_async_copy(feat_hbm.at[csrc.at[pl.ds(0, 16)]],
                              fbuf.at[slot], gsem.at[slot]).wait()
        pltpu.make_async_copy(elrs_hbm.at[csrc.at[pl.ds(0, 16)]],
                              esbuf.at[slot], gsem.at[slot]).wait()
        pltpu.make_async_copy(elrd_hbm.at[cdst.at[pl.ds(0, 16)]],
                              edbuf.at[slot], gsem.at[slot]).wait()

      def wait_scatter(slot):
        pltpu.make_async_copy(fbuf.at[slot], acc.at[didx.at[slot]],
                              ssem.at[slot]).wait()
        pltpu.make_async_copy(wbuf.at[slot], den.at[didx.at[slot]],
                              ssem.at[slot]).wait()

      pl.when(nchunks > 0)(lambda: fire(0, 0))

      def chunk(ci, c):
        b = lax.rem(ci, 3)
        nb = lax.rem(ci + 1, 3)
        pl.when(ci + 1 < nchunks)(lambda: fire(ci + 1, nb))
        pl.when(ci >= 3)(lambda: wait_scatter(b))
        base = pl.multiple_of(ci * 16, 16)
        dv = cdst[pl.ds(base, 16)]
        didx[b, :] = dv - lo
        wait_gather(b)
        for e in range(16):
          s = esbuf[b, e, :] + edbuf[b, e, :]
          wbuf[b, e, :] = jnp.exp(jnp.where(s > 0, s, s * 0.2))
        for e in range(16):
          w = wbuf[b, e, :]
          for j in range(32):
            sl = pl.ds(j * 16, 16)
            fbuf[b, e, sl] = fbuf[b, e, sl] * w
        pltpu.async_copy(fbuf.at[b], acc.at[didx.at[b]], ssem.at[b],
                         add=True)
        pltpu.async_copy(wbuf.at[b], den.at[didx.at[b]], ssem.at[b],
                         add=True)
        return c

      lax.fori_loop(0, nchunks, chunk, 0)
      for d in range(3):
        pl.when(nchunks > d)(
            functools.partial(wait_scatter, lax.rem(nchunks - 1 - d + 3, 3)))
      return hcarry

    lax.fori_loop(0, NSUB0, half, 0)
    plsc.subcore_barrier()

    pltpu.sync_copy(acc.at[pl.ds(t * RPT0, RPT0)],
                    num_out.at[ri, pl.ds(t * RPT0, RPT0)])
    pltpu.sync_copy(den.at[pl.ds(t * RPT0, RPT0)],
                    den_out.at[ri, pl.ds(t * RPT0, RPT0)])
    plsc.subcore_barrier()
    return carry

  lax.fori_loop(0, 2, do_pass, 0)


def _edge1_body(feat1_hbm, e1s_hbm, e1d_hbm, src_hbm, dst_hbm,
                num_out, den_out,
                stage_src, stage_dst, didx,
                fbuf, esbuf, edbuf, wbuf, zbuf48, zbufd,
                acc, den, gsem, ssem):
  cid = lax.axis_index("c")
  t = lax.axis_index("s")
  w32 = t * 2 + cid
  iota16 = lax.iota(I32, 16)
  zero16 = jnp.zeros((16,), F32)

  def _zb(r, c):
    for j in range(3):
      zbuf48[r, pl.ds(j * 16, 16)] = zero16
    zbufd[r, :] = zero16
    return c
  lax.fori_loop(0, RPT1, _zb, 0)
  pltpu.sync_copy(zbuf48, acc.at[pl.ds(t * RPT1, RPT1)])
  pltpu.sync_copy(zbufd, den.at[pl.ds(t * RPT1, RPT1)])

  base_e = pl.multiple_of(w32 * EPT1, 16)
  pltpu.sync_copy(src_hbm.at[pl.ds(base_e, EPT1)], stage_src.at[pl.ds(0, EPT1)])
  pltpu.sync_copy(dst_hbm.at[pl.ds(base_e, EPT1)], stage_dst.at[pl.ds(0, EPT1)])
  for pp in range(3):
    stage_src[pl.ds(EPT1 + pp * 16, 16)] = iota16
    stage_dst[pl.ds(EPT1 + pp * 16, 16)] = N + iota16
  plsc.subcore_barrier()

  def fire(q, slot):
    qb = pl.multiple_of(q * 64, 64)
    cs = stage_src.at[pl.ds(qb, 64)]
    cd = stage_dst.at[pl.ds(qb, 64)]
    pltpu.async_copy(feat1_hbm.at[cs], fbuf.at[slot], gsem.at[slot])
    pltpu.async_copy(e1s_hbm.at[cs], esbuf.at[slot], gsem.at[slot])
    pltpu.async_copy(e1d_hbm.at[cd], edbuf.at[slot], gsem.at[slot])

  def wait_gather(slot):
    pltpu.make_async_copy(feat1_hbm.at[stage_src.at[pl.ds(0, 64)]],
                          fbuf.at[slot], gsem.at[slot]).wait()
    pltpu.make_async_copy(e1s_hbm.at[stage_src.at[pl.ds(0, 64)]],
                          esbuf.at[slot], gsem.at[slot]).wait()
    pltpu.make_async_copy(e1d_hbm.at[stage_dst.at[pl.ds(0, 64)]],
                          edbuf.at[slot], gsem.at[slot]).wait()

  def wait_scatter(slot):
    pltpu.make_async_copy(fbuf.at[slot], acc.at[didx.at[slot]],
                          ssem.at[slot]).wait()
    pltpu.make_async_copy(wbuf.at[slot], den.at[didx.at[slot]],
                          ssem.at[slot]).wait()

  fire(0, 0)

  def chunk(q, c):
    b = lax.rem(q, 3)
    nb = lax.rem(q + 1, 3)
    pl.when(q + 1 < NCH1)(lambda: fire(q + 1, nb))
    pl.when(q >= 3)(lambda: wait_scatter(b))
    qb = pl.multiple_of(q * 64, 64)
    for pp in range(4):
      didx[b, pl.ds(pp * 16, 16)] = stage_dst[pl.ds(qb + pp * 16, 16)]
    wait_gather(b)
    for e in range(64):
      s = esbuf[b, e, :] + edbuf[b, e, :]
      wbuf[b, e, :] = jnp.exp(jnp.where(s > 0, s, s * 0.2))
    for e in range(64):
      w = wbuf[b, e, :]
      for j in range(3):
        sl = pl.ds(j * 16, 16)
        fbuf[b, e, sl] = fbuf[b, e, sl] * w
    pltpu.async_copy(fbuf.at[b], acc.at[didx.at[b]], ssem.at[b], add=True)
    pltpu.async_copy(wbuf.at[b], den.at[didx.at[b]], ssem.at[b], add=True)
    return c

  lax.fori_loop(0, NCH1, chunk, 0)
  for d in range(3):
    wait_scatter((NCH1 - 1 - d) % 3)
  plsc.subcore_barrier()
  pltpu.sync_copy(acc.at[pl.ds(t * RPT1, RPT1)],
                  num_out.at[cid, pl.ds(t * RPT1, RPT1)])
  pltpu.sync_copy(den.at[pl.ds(t * RPT1, RPT1)],
                  den_out.at[cid, pl.ds(t * RPT1, RPT1)])


_sc_mesh = plsc.VectorSubcoreMesh(core_axis_name="c", subcore_axis_name="s")
_sc_params = pltpu.CompilerParams(
    needs_layout_passes=False, use_tc_tiling_on_sc=False)

_edge0 = pl.kernel(
    _edge0_body,
    compiler_params=_sc_params,
    out_type=(jax.ShapeDtypeStruct((NRANGE, ACCR, 512), F32),
              jax.ShapeDtypeStruct((NRANGE, ACCR, 16), F32)),
    mesh=_sc_mesh,
    scratch_types=[
        pltpu.VMEM((BLKE,), I32),
        pltpu.VMEM((BLKE,), I32),
        pltpu.VMEM((BLKE + 16,), I32),
        pltpu.VMEM((BLKE + 16,), I32),
        pltpu.VMEM((3, 16), I32),
        pltpu.VMEM((3, 16, 512), F32),
        pltpu.VMEM((3, 16, 16), F32),
        pltpu.VMEM((3, 16, 16), F32),
        pltpu.VMEM((3, 16, 16), F32),
        pltpu.VMEM((8, 512), F32),
        pltpu.VMEM((16, 16), F32),
        pltpu.VMEM_SHARED((ACCR, 512), F32),
        pltpu.VMEM_SHARED((ACCR, 16), F32),
        pltpu.SemaphoreType.DMA((3,)),
        pltpu.SemaphoreType.DMA((3,)),
    ])

_edge1 = pl.kernel(
    _edge1_body,
    compiler_params=_sc_params,
    out_type=(jax.ShapeDtypeStruct((2, ACC1R, 48), F32),
              jax.ShapeDtypeStruct((2, ACC1R, 16), F32)),
    mesh=_sc_mesh,
    scratch_types=[
        pltpu.VMEM((EPT1 + 48,), I32),
        pltpu.VMEM((EPT1 + 48,), I32),
        pltpu.VMEM((3, 64), I32),
        pltpu.VMEM((3, 64, 48), F32),
        pltpu.VMEM((3, 64, 16), F32),
        pltpu.VMEM((3, 64, 16), F32),
        pltpu.VMEM((3, 64, 16), F32),
        pltpu.VMEM((RPT1, 48), F32),
        pltpu.VMEM((RPT1, 16), F32),
        pltpu.VMEM_SHARED((ACC1R, 48), F32),
        pltpu.VMEM_SHARED((ACC1R, 16), F32),
        pltpu.SemaphoreType.DMA((3,)),
        pltpu.SemaphoreType.DMA((3,)),
    ])


def _dense0_body(x_ref, w_ref, a_ref, b_ref, feat_ref, es_ref, ed_ref):
  f = jnp.dot(x_ref[...], w_ref[...], preferred_element_type=F32)
  feat_ref[...] = f
  es_ref[...] = jnp.dot(f, a_ref[...], preferred_element_type=F32)
  ed_ref[...] = jnp.dot(f, b_ref[...], preferred_element_type=F32)


_dense0 = pl.pallas_call(
    _dense0_body,
    grid=(10,),
    in_specs=[
        pl.BlockSpec((1000, 128), lambda i: (i, 0)),
        pl.BlockSpec((128, 512), lambda i: (0, 0)),
        pl.BlockSpec((512, 16), lambda i: (0, 0)),
        pl.BlockSpec((512, 16), lambda i: (0, 0)),
    ],
    out_specs=[
        pl.BlockSpec((1000, 512), lambda i: (i, 0)),
        pl.BlockSpec((1000, 16), lambda i: (i, 0)),
        pl.BlockSpec((1000, 16), lambda i: (i, 0)),
    ],
    out_shape=[
        jax.ShapeDtypeStruct((N, 512), F32),
        jax.ShapeDtypeStruct((N, 16), F32),
        jax.ShapeDtypeStruct((N, 16), F32),
    ])


def _node0_body(num_ref, den_ref, b0_ref, s_ref, w1_ref, wres_ref, a1_ref,
                b1a_ref, feat1_ref, res_ref, e1s_ref, e1d_ref):
  nb = num_ref[...]
  db = den_ref[...]
  dbig = jnp.dot(db, s_ref[...], preferred_element_type=F32)
  pre = nb / (dbig + 1e-9) + b0_ref[...]
  h = jnp.where(pre > 0, pre, jnp.exp(pre) - 1.0)
  feat1_ref[...] = jnp.dot(h, w1_ref[...], preferred_element_type=F32)
  res_ref[...] = jnp.dot(h, wres_ref[...], preferred_element_type=F32)
  e1s_ref[...] = jnp.dot(h, a1_ref[...], preferred_element_type=F32)
  e1d_ref[...] = jnp.dot(h, b1a_ref[...], preferred_element_type=F32)


_node0 = pl.pallas_call(
    _node0_body,
    grid=(10,),
    in_specs=[
        pl.BlockSpec((1000, 512), lambda i: (i, 0)),
        pl.BlockSpec((1000, 16), lambda i: (i, 0)),
        pl.BlockSpec((1, 512), lambda i: (0, 0)),
        pl.BlockSpec((16, 512), lambda i: (0, 0)),
        pl.BlockSpec((512, 48), lambda i: (0, 0)),
        pl.BlockSpec((512, 48), lambda i: (0, 0)),
        pl.BlockSpec((512, 16), lambda i: (0, 0)),
        pl.BlockSpec((512, 16), lambda i: (0, 0)),
    ],
    out_specs=[
        pl.BlockSpec((1000, 48), lambda i: (i, 0)),
        pl.BlockSpec((1000, 48), lambda i: (i, 0)),
        pl.BlockSpec((1000, 16), lambda i: (i, 0)),
        pl.BlockSpec((1000, 16), lambda i: (i, 0)),
    ],
    out_shape=[
        jax.ShapeDtypeStruct((N, 48), F32),
        jax.ShapeDtypeStruct((N, 48), F32),
        jax.ShapeDtypeStruct((N, 16), F32),
        jax.ShapeDtypeStruct((N, 16), F32),
    ])


def _final_body(n1_ref, d1_ref, res_ref, b1_ref, s1_ref, out_ref):
  ns = n1_ref[0] + n1_ref[1]
  dsum = d1_ref[0] + d1_ref[1]
  dbig = jnp.dot(dsum, s1_ref[...], preferred_element_type=F32)
  out_ref[...] = ns / (dbig + 1e-9) + res_ref[...] + b1_ref[...]


_final = pl.pallas_call(
    _final_body,
    grid=(10,),
    in_specs=[
        pl.BlockSpec((2, 1000, 48), lambda i: (0, i, 0)),
        pl.BlockSpec((2, 1000, 16), lambda i: (0, i, 0)),
        pl.BlockSpec((1000, 48), lambda i: (i, 0)),
        pl.BlockSpec((1, 48), lambda i: (0, 0)),
        pl.BlockSpec((16, 48), lambda i: (0, 0)),
    ],
    out_specs=pl.BlockSpec((1000, 48), lambda i: (i, 0)),
    out_shape=jax.ShapeDtypeStruct((N, 48), F32))


def kernel(x, edge_index0, edge_index1, W0, al0, ar0, b0, W1, al1, ar1, b1,
           Wres):
  W0_il = W0.reshape(128, 8, 64).transpose(0, 2, 1).reshape(128, 512)
  al_flat = al0.T.reshape(512)
  ar_flat = ar0.T.reshape(512)
  eyea = jnp.asarray(_EYEA)
  Ail = al_flat[:, None] * eyea
  Bil = ar_flat[:, None] * eyea
  b0_il = b0.reshape(8, 64).T.reshape(512)[None, :]
  S_il = jnp.asarray(_SIL)
  W1p = jnp.pad(W1, ((0, 0), (0, 8))).reshape(8, 64, 48).transpose(1, 0, 2).reshape(512, 48)
  Wresp = jnp.pad(Wres, ((0, 0), (0, 8))).reshape(8, 64, 48).transpose(1, 0, 2).reshape(512, 48)
  a1c = (W1 @ al1[0]).reshape(8, 64).T.reshape(512)
  b1c = (W1 @ ar1[0]).reshape(8, 64).T.reshape(512)
  A1 = jnp.tile(a1c[:, None], (1, 16)).astype(F32)
  B1 = jnp.tile(b1c[:, None], (1, 16)).astype(F32)
  b1p = jnp.pad(b1, (0, 8))[None, :]
  S1 = jnp.asarray(_S1)

  src0, dst0 = edge_index0[0], edge_index0[1]
  src1, dst1 = edge_index1[0], edge_index1[1]

  feat, elrS, elrD = _dense0(x, W0_il, Ail, Bil)
  elrD_p = jnp.pad(elrD, ((0, NPAD0 - N), (0, 0)))
  num0, den0 = _edge0(feat, elrS, elrD_p, src0, dst0)
  num0f = num0[:, :NQ, :].reshape(NRANGE * NQ, 512)[:N]
  den0f = den0[:, :NQ, :].reshape(NRANGE * NQ, 16)[:N]
  feat1, res, e1s, e1d = _node0(num0f, den0f, b0_il, S_il, W1p, Wresp, A1, B1)
  e1d_p = jnp.pad(e1d, ((0, 16), (0, 0)))
  num1, den1 = _edge1(feat1, e1s, e1d_p, src1, dst1)
  out48 = _final(num1, den1, res, b1p, S1)
  return out48[:, :40]
